# Initial kernel scaffold; baseline (speedup 1.0000x reference)
import jax
import jax.numpy as jnp
from jax.experimental import pallas as pl


def kernel(inputs, embed_table, W, b_l1, bias):
    raise NotImplementedError("write your pallas kernel here")


# speedup vs baseline: 23.2636x; 23.2636x over previous
import functools

import jax
import jax.numpy as jnp
from jax import lax
from jax.experimental import pallas as pl
from jax.experimental.pallas import tpu as pltpu
from jax.experimental.pallas import tpu_sc as plsc

B = 16384
L = 50
D = 32
OUT = 5

NW = 32
ROWS_PER_W = B // NW
PAIRS_PER_W = ROWS_PER_W // 2
IDX_W = 2 * L + 4


def _sc_sums(idx_pad, table):
    mesh = plsc.VectorSubcoreMesh(core_axis_name="c", subcore_axis_name="s")

    @functools.partial(
        pl.kernel,
        mesh=mesh,
        out_type=jax.ShapeDtypeStruct((B, D), jnp.float32),
        scratch_types=[
            pltpu.VMEM((PAIRS_PER_W, IDX_W), jnp.int32),
            pltpu.VMEM((IDX_W, D), jnp.float32),
            pltpu.VMEM((IDX_W, D), jnp.float32),
            pltpu.VMEM((ROWS_PER_W, D), jnp.float32),
            pltpu.SemaphoreType.DMA,
            pltpu.SemaphoreType.DMA,
        ],
        compiler_params=pltpu.CompilerParams(use_tc_tiling_on_sc=False),
    )
    def k(idx_hbm, table_hbm, out_hbm, idx_v, buf0, buf1, out_v, sem0, sem1):
        w = lax.axis_index("s") * 2 + lax.axis_index("c")
        pltpu.sync_copy(idx_hbm.at[pl.ds(w * PAIRS_PER_W, PAIRS_PER_W)], idx_v)
        bufs = (buf0, buf1)
        sems = (sem0, sem1)
        pltpu.async_copy(table_hbm.at[idx_v.at[0]], buf0, sem0)
        pltpu.async_copy(table_hbm.at[idx_v.at[1]], buf1, sem1)

        def accum(buf, g):
            for r in range(2):
                a0 = buf[r * L, pl.ds(0, 16)]
                a1 = buf[r * L, pl.ds(16, 16)]
                for j in range(1, L):
                    a0 = a0 + buf[r * L + j, pl.ds(0, 16)]
                    a1 = a1 + buf[r * L + j, pl.ds(16, 16)]
                out_v[2 * g + r, pl.ds(0, 16)] = a0
                out_v[2 * g + r, pl.ds(16, 16)] = a1

        def body(i, carry):
            for b in range(2):
                g = 2 * i + b
                pltpu.make_async_copy(
                    table_hbm.at[idx_v.at[g]], bufs[b], sems[b]).wait()
                accum(bufs[b], g)

                @pl.when(g + 2 < PAIRS_PER_W)
                def _():
                    pltpu.async_copy(
                        table_hbm.at[idx_v.at[g + 2]], bufs[b], sems[b])
            return carry

        lax.fori_loop(0, PAIRS_PER_W // 2, body, 0)
        pltpu.sync_copy(out_v, out_hbm.at[pl.ds(w * ROWS_PER_W, ROWS_PER_W)])

    return k(idx_pad, table)


def _project(sums, wt_pad, c_pad):
    blk = 2048

    def pk(x_ref, w_ref, c_ref, o_ref):
        o_ref[...] = (
            jnp.dot(x_ref[...], w_ref[...], preferred_element_type=jnp.float32)
            + c_ref[...]
        )

    return pl.pallas_call(
        pk,
        grid=(B // blk,),
        in_specs=[
            pl.BlockSpec((blk, D), lambda i: (i, 0)),
            pl.BlockSpec((D, 8), lambda i: (0, 0)),
            pl.BlockSpec((1, 8), lambda i: (0, 0)),
        ],
        out_specs=pl.BlockSpec((blk, 8), lambda i: (i, 0)),
        out_shape=jax.ShapeDtypeStruct((B, 8), jnp.float32),
    )(sums, wt_pad, c_pad)


def kernel(inputs, embed_table, W, b_l1, bias):
    idx = inputs.astype(jnp.int32).reshape(NW * PAIRS_PER_W, 2 * L)
    idx_pad = jnp.pad(idx, ((0, 0), (0, IDX_W - 2 * L)))
    sums = _sc_sums(idx_pad, embed_table)
    wt_pad = jnp.pad(W.T, ((0, 0), (0, 8 - OUT)))
    c_pad = jnp.pad(L * b_l1 + bias, (0, 8 - OUT)).reshape(1, 8)
    return _project(sums, wt_pad, c_pad)[:, :OUT]
